# baseline (speedup 1.0000x reference)
import jax
import jax.numpy as jnp
from jax.experimental import pallas as pl
from jax.experimental.pallas import tpu as pltpu

N = 10000
D = 128
F = 8
H = 64
K = 16
BLK = 5000
FK = F * K
CW = 4 * FK


def _fused_kernel(x_ref, w1_ref, b1_ref, w2_ref, b2_ref, raw_ref, outw_ref,
                  outb_ref, out_ref, wperm_ref):
    f32 = jnp.float32

    @pl.when(pl.program_id(0) == 0)
    def _():
        i = jax.lax.broadcasted_iota(jnp.int32, (CW, CW), 0)
        j = jax.lax.broadcasted_iota(jnp.int32, (CW, CW), 1)
        perm_i = (i % FK) // K * (4 * K) + (i // FK) * K + (i % K)
        P = (perm_i == j).astype(f32)
        wperm_ref[...] = jnp.dot(P, outw_ref[D:, :],
                                 preferred_element_type=f32)

    sf = jax.lax.broadcasted_iota(jnp.int32, (F, FK), 0)
    sj = jax.lax.broadcasted_iota(jnp.int32, (F, FK), 1)
    S = (sj // K == sf).astype(f32)

    mk = jax.lax.broadcasted_iota(jnp.int32, (K, FK), 0)
    mj = jax.lax.broadcasted_iota(jnp.int32, (K, FK), 1)
    M16 = (mj % K == mk).astype(f32)
    tiled = jnp.dot(raw_ref[...], M16, preferred_element_type=f32)
    t_tri = tiled[0:1, :]
    c0 = tiled[1:2, :]
    c1 = tiled[2:3, :]
    ls = tiled[3:4, :] + tiled[4:5, :]
    bl = tiled[5:6, :]
    ch0 = tiled[6:7, :]
    ch1 = tiled[7:8, :]
    r = jnp.abs(tiled[8:9, :])

    xb = x_ref[...]
    h = jnp.maximum(
        jnp.dot(xb, w1_ref[...], preferred_element_type=f32)
        + b1_ref[...], 0.0)
    w2s = jnp.dot(w2_ref[...], S, preferred_element_type=f32)
    b2e = jnp.dot(b2_ref[...], S, preferred_element_type=f32)
    vE = jnp.dot(h, w2s, preferred_element_type=f32) + b2e

    tri = jnp.maximum(vE - jnp.abs(vE - t_tri), 0.0)
    g0 = vE - c0
    g1 = vE - c1
    gau = jnp.exp((g0 * g0 + g1 * g1) * -0.5)
    lin = vE * ls + bl
    d = jnp.abs(vE - ch0) + jnp.abs(vE - ch1)
    hat = 1.0 / (1.0 + d) - 1.0 / (1.0 + jnp.abs(r - d))

    acc = jnp.dot(xb, outw_ref[0:D, :], preferred_element_type=f32)
    acc += jnp.dot(tri, wperm_ref[0:FK, :], preferred_element_type=f32)
    acc += jnp.dot(gau, wperm_ref[FK:2 * FK, :], preferred_element_type=f32)
    acc += jnp.dot(lin, wperm_ref[2 * FK:3 * FK, :], preferred_element_type=f32)
    acc += jnp.dot(hat, wperm_ref[3 * FK:4 * FK, :], preferred_element_type=f32)
    out_ref[...] = jnp.maximum(acc + outb_ref[...], 0.0)


def kernel(x, edge_index, batch_idx, edge_slices, W1, b1, W2, b2, t_tri,
           c_gauss, w_line, b_line, c_hat, r_hat, out_W, out_b):
    raw = jnp.concatenate([
        t_tri[None, :],
        c_gauss.T,
        w_line,
        b_line[None, :],
        c_hat.T,
        jnp.broadcast_to(r_hat[:, None], (1, K)),
    ], axis=0)

    grid = (N // BLK,)
    return pl.pallas_call(
        _fused_kernel,
        grid=grid,
        in_specs=[
            pl.BlockSpec((BLK, D), lambda i: (i, 0)),
            pl.BlockSpec((D, H), lambda i: (0, 0)),
            pl.BlockSpec((1, H), lambda i: (0, 0)),
            pl.BlockSpec((H, F), lambda i: (0, 0)),
            pl.BlockSpec((1, F), lambda i: (0, 0)),
            pl.BlockSpec((9, K), lambda i: (0, 0)),
            pl.BlockSpec((D + CW, D), lambda i: (0, 0)),
            pl.BlockSpec((1, D), lambda i: (0, 0)),
        ],
        out_specs=pl.BlockSpec((BLK, D), lambda i: (i, 0)),
        out_shape=jax.ShapeDtypeStruct((N, D), jnp.float32),
        scratch_shapes=[pltpu.VMEM((CW, D), jnp.float32)],
    )(x, W1, b1[None, :], W2, b2[None, :], raw, out_W, out_b[None, :])


# speedup vs baseline: 1.1612x; 1.1612x over previous
import jax
import jax.numpy as jnp
from jax.experimental import pallas as pl
from jax.experimental.pallas import tpu as pltpu

N = 10000
D = 128
F = 8
H = 64
K = 16
BLK = 2000
FK = F * K
CW = 4 * FK


def _fused_kernel(x_ref, w1_ref, b1_ref, w2_ref, b2_ref, raw_ref, outw_ref,
                  outb_ref, out_ref, wperm_ref, cst_ref, w2s_ref, b2e_ref):
    f32 = jnp.float32

    @pl.when(pl.program_id(0) == 0)
    def _():
        i = jax.lax.broadcasted_iota(jnp.int32, (CW, CW), 0)
        j = jax.lax.broadcasted_iota(jnp.int32, (CW, CW), 1)
        perm_i = (i % FK) // K * (4 * K) + (i // FK) * K + (i % K)
        P = (perm_i == j).astype(f32)
        wperm_ref[...] = jnp.dot(P, outw_ref[D:, :],
                                 preferred_element_type=f32)

    @pl.when(pl.program_id(0) == 0)
    def _():
        sf = jax.lax.broadcasted_iota(jnp.int32, (F, FK), 0)
        sj = jax.lax.broadcasted_iota(jnp.int32, (F, FK), 1)
        S = (sj // K == sf).astype(f32)
        w2s_ref[...] = jnp.dot(w2_ref[...], S, preferred_element_type=f32)
        b2e_ref[...] = jnp.dot(b2_ref[...], S, preferred_element_type=f32)

        mk = jax.lax.broadcasted_iota(jnp.int32, (K, FK), 0)
        mj = jax.lax.broadcasted_iota(jnp.int32, (K, FK), 1)
        M16 = (mj % K == mk).astype(f32)
        cst_ref[...] = jnp.dot(raw_ref[...], M16, preferred_element_type=f32)

    t_tri = cst_ref[0:1, :]
    c0 = cst_ref[1:2, :]
    c1 = cst_ref[2:3, :]
    ls = cst_ref[3:4, :] + cst_ref[4:5, :]
    bl = cst_ref[5:6, :]
    ch0 = cst_ref[6:7, :]
    ch1 = cst_ref[7:8, :]
    r = jnp.abs(cst_ref[8:9, :])

    xb = x_ref[...]
    h = jnp.maximum(
        jnp.dot(xb, w1_ref[...], preferred_element_type=f32)
        + b1_ref[...], 0.0)
    vE = (jnp.dot(h, w2s_ref[...], preferred_element_type=f32)
          + b2e_ref[...])

    tri = jnp.maximum(vE - jnp.abs(vE - t_tri), 0.0)
    g0 = vE - c0
    g1 = vE - c1
    gau = jnp.exp((g0 * g0 + g1 * g1) * -0.5)
    lin = vE * ls + bl
    d = jnp.abs(vE - ch0) + jnp.abs(vE - ch1)
    hat = 1.0 / (1.0 + d) - 1.0 / (1.0 + jnp.abs(r - d))

    acc = jnp.dot(xb, outw_ref[0:D, :], preferred_element_type=f32)
    acc += jnp.dot(tri, wperm_ref[0:FK, :], preferred_element_type=f32)
    acc += jnp.dot(gau, wperm_ref[FK:2 * FK, :], preferred_element_type=f32)
    acc += jnp.dot(lin, wperm_ref[2 * FK:3 * FK, :], preferred_element_type=f32)
    acc += jnp.dot(hat, wperm_ref[3 * FK:4 * FK, :], preferred_element_type=f32)
    out_ref[...] = jnp.maximum(acc + outb_ref[...], 0.0)


def kernel(x, edge_index, batch_idx, edge_slices, W1, b1, W2, b2, t_tri,
           c_gauss, w_line, b_line, c_hat, r_hat, out_W, out_b):
    raw = jnp.concatenate([
        t_tri[None, :],
        c_gauss.T,
        w_line,
        b_line[None, :],
        c_hat.T,
        jnp.broadcast_to(r_hat[:, None], (1, K)),
    ], axis=0)

    grid = (N // BLK,)
    return pl.pallas_call(
        _fused_kernel,
        grid=grid,
        in_specs=[
            pl.BlockSpec((BLK, D), lambda i: (i, 0)),
            pl.BlockSpec((D, H), lambda i: (0, 0)),
            pl.BlockSpec((1, H), lambda i: (0, 0)),
            pl.BlockSpec((H, F), lambda i: (0, 0)),
            pl.BlockSpec((1, F), lambda i: (0, 0)),
            pl.BlockSpec((9, K), lambda i: (0, 0)),
            pl.BlockSpec((D + CW, D), lambda i: (0, 0)),
            pl.BlockSpec((1, D), lambda i: (0, 0)),
        ],
        out_specs=pl.BlockSpec((BLK, D), lambda i: (i, 0)),
        out_shape=jax.ShapeDtypeStruct((N, D), jnp.float32),
        scratch_shapes=[pltpu.VMEM((CW, D), jnp.float32),
                        pltpu.VMEM((9, FK), jnp.float32),
                        pltpu.VMEM((H, FK), jnp.float32),
                        pltpu.VMEM((1, FK), jnp.float32)],
    )(x, W1, b1[None, :], W2, b2[None, :], raw, out_W, out_b[None, :])
